# baseline (speedup 1.0000x reference)
import functools

import jax
import jax.numpy as jnp
from jax import lax
from jax.experimental import pallas as pl
from jax.experimental.pallas import tpu as pltpu
from jax.experimental.pallas import tpu_sc as plsc

_N = 4096
_VOCAB = 100000
_HV = _VOCAB // 2
_EDIM = 64
_WFEAT = 64
_LABELS = 2
_SLOPE = 0.01
_BR = 256
_NSTRIPS = _N // _BR
_BW = 2000
_NWH = _HV // _BW

_NC = 2
_NS = 16
_NW = _NC * _NS
_BPW = _N // _NW


def _wh_all_body(emb_ref, w_ref, out_ref):
    j = pl.program_id(1)
    wh = jnp.dot(emb_ref[...], w_ref[...],
                 preferred_element_type=jnp.float32)

    @pl.when(j == 0)
    def _lo():
        out_ref[:, :_WFEAT] = wh

    @pl.when(j == 1)
    def _hi():
        out_ref[:, _WFEAT:] = wh


def _wh_all(emb, W):
    return pl.pallas_call(
        _wh_all_body,
        grid=(_NWH, 2),
        in_specs=[
            pl.BlockSpec((_BW, _EDIM), lambda i, j: (j * _NWH + i, 0)),
            pl.BlockSpec((_EDIM, _WFEAT), lambda i, j: (0, 0)),
        ],
        out_specs=pl.BlockSpec((_BW, 2 * _WFEAT), lambda i, j: (i, 0)),
        out_shape=jax.ShapeDtypeStruct((_HV, 2 * _WFEAT), jnp.float32),
        compiler_params=pltpu.CompilerParams(
            dimension_semantics=("arbitrary", "arbitrary")),
    )(emb, W)


def _sc_gather_body(table_hbm, idx_hbm, out_hbm, idx_v, rows_v, sem):
    wid = lax.axis_index("s") * _NC + lax.axis_index("c")
    base = wid * _BPW
    pltpu.sync_copy(idx_hbm.at[pl.ds(base, _BPW)], idx_v)
    pltpu.async_copy(table_hbm.at[idx_v], rows_v, sem).wait()
    pltpu.sync_copy(rows_v, out_hbm.at[pl.ds(base, _BPW)])


def _sc_gather(table2, idx_mod):
    mesh = plsc.VectorSubcoreMesh(core_axis_name="c", subcore_axis_name="s")
    fn = functools.partial(
        pl.kernel,
        mesh=mesh,
        out_type=jax.ShapeDtypeStruct((_N, 2 * _WFEAT), jnp.float32),
        scratch_types=[
            pltpu.VMEM((_BPW,), jnp.int32),
            pltpu.VMEM((_BPW, 2 * _WFEAT), jnp.float32),
            pltpu.SemaphoreType.DMA,
        ],
    )(_sc_gather_body)
    return fn(table2, idx_mod)


def _tc_body(adj_ref, pairs_ref, half_ref, a1_ref, a2r_ref, wc_ref,
             bc_ref, att_ref, sent_ref, pool_ref, label_ref, wh_s, f2r_s,
             pool_s):
    i = pl.program_id(0)

    @pl.when(i == 0)
    def _init():
        pairs = pairs_ref[...]
        wh = jnp.where(half_ref[...] > 0,
                       lax.slice(pairs, (0, _WFEAT), (_N, 2 * _WFEAT)),
                       lax.slice(pairs, (0, 0), (_N, _WFEAT)))
        wh_s[...] = wh
        f2r_s[...] = lax.dot_general(
            a2r_ref[...], wh, (((1,), (1,)), ((), ())),
            preferred_element_type=jnp.float32)
        pool_s[...] = jnp.zeros_like(pool_s)

    row0 = pl.multiple_of(i * _BR, _BR)
    wh_strip = wh_s[pl.ds(row0, _BR), :]
    f1 = jnp.dot(wh_strip, a1_ref[...],
                 preferred_element_type=jnp.float32)
    e = f1 + f2r_s[...]
    e = jnp.maximum(e, _SLOPE * e)
    e = jnp.where(adj_ref[...] > 0, e, jnp.float32(-9e15))
    m = jnp.max(e, axis=1, keepdims=True)
    p = jnp.exp(e - m)
    s = jnp.sum(p, axis=1, keepdims=True)
    att = p / s
    att_ref[...] = att
    h = jnp.dot(att, wh_s[...], preferred_element_type=jnp.float32)
    sent = jnp.where(h > 0, h, jnp.exp(h) - 1.0)
    sent_ref[...] = sent
    pool_s[...] += jnp.sum(sent, axis=0, keepdims=True)

    @pl.when(i == _NSTRIPS - 1)
    def _fin():
        pool = pool_s[...] / jnp.float32(_N)
        pool_ref[...] = pool
        logits = jnp.dot(pool, wc_ref[...],
                         preferred_element_type=jnp.float32) + bc_ref[...]
        lm = jnp.max(logits, axis=1, keepdims=True)
        ex = jnp.exp(logits - lm)
        label_ref[...] = ex / jnp.sum(ex, axis=1, keepdims=True)


def _tc_main(adj, pairs, half, a1, a2r, Wc, bcr):
    return pl.pallas_call(
        _tc_body,
        grid=(_NSTRIPS,),
        in_specs=[
            pl.BlockSpec((_BR, _N), lambda i: (i, 0)),
            pl.BlockSpec((_N, 2 * _WFEAT), lambda i: (0, 0)),
            pl.BlockSpec((_N, 1), lambda i: (0, 0)),
            pl.BlockSpec((_WFEAT, 1), lambda i: (0, 0)),
            pl.BlockSpec((1, _WFEAT), lambda i: (0, 0)),
            pl.BlockSpec((_WFEAT, _LABELS), lambda i: (0, 0)),
            pl.BlockSpec((1, _LABELS), lambda i: (0, 0)),
        ],
        out_specs=[
            pl.BlockSpec((_BR, _N), lambda i: (i, 0)),
            pl.BlockSpec((_BR, _WFEAT), lambda i: (i, 0)),
            pl.BlockSpec((1, _WFEAT), lambda i: (0, 0)),
            pl.BlockSpec((1, _LABELS), lambda i: (0, 0)),
        ],
        out_shape=[
            jax.ShapeDtypeStruct((_N, _N), jnp.float32),
            jax.ShapeDtypeStruct((_N, _WFEAT), jnp.float32),
            jax.ShapeDtypeStruct((1, _WFEAT), jnp.float32),
            jax.ShapeDtypeStruct((1, _LABELS), jnp.float32),
        ],
        scratch_shapes=[
            pltpu.VMEM((_N, _WFEAT), jnp.float32),
            pltpu.VMEM((1, _N), jnp.float32),
            pltpu.VMEM((1, _WFEAT), jnp.float32),
        ],
        compiler_params=pltpu.CompilerParams(
            dimension_semantics=("arbitrary",)),
    )(adj, pairs, half, a1, a2r, Wc, bcr)


def kernel(inSen, adj, emb, W, a, Wc, bc):
    idx = inSen.astype(jnp.int32)
    table2 = _wh_all(emb, W)
    pairs = _sc_gather(table2, idx % _HV)
    half = (idx // _HV).astype(jnp.float32).reshape(_N, 1)
    a1 = a[:_WFEAT]
    a2r = a[_WFEAT:].reshape(1, _WFEAT)
    bcr = bc.reshape(1, _LABELS)
    att, sent, pool, label = _tc_main(adj, pairs, half, a1, a2r, Wc, bcr)
    return (pool.reshape(_WFEAT), att, sent, label.reshape(_LABELS))


# speedup vs baseline: 1.1634x; 1.1634x over previous
import functools

import jax
import jax.numpy as jnp
from jax import lax
from jax.experimental import pallas as pl
from jax.experimental.pallas import tpu as pltpu
from jax.experimental.pallas import tpu_sc as plsc

_N = 4096
_VOCAB = 100000
_HV = _VOCAB // 2
_EDIM = 64
_WFEAT = 64
_LABELS = 2
_SLOPE = 0.01
_BR = 256
_NSTRIPS = _N // _BR
_BW = 2000
_NWH = _HV // _BW

_NC = 2
_NS = 16
_NW = _NC * _NS
_BPW = _N // _NW


def _wh_all_body(emb_ref, w_ref, out_ref):
    wh = jnp.dot(emb_ref[...], w_ref[...],
                 preferred_element_type=jnp.float32)
    out_ref[...] = jnp.concatenate(
        [lax.slice(wh, (0, 0), (_BW, _WFEAT)),
         lax.slice(wh, (_BW, 0), (2 * _BW, _WFEAT))], axis=1)


def _wh_all(emb, W):
    return pl.pallas_call(
        _wh_all_body,
        grid=(_NWH,),
        in_specs=[
            pl.BlockSpec((2 * _BW, _EDIM), lambda i: (i, 0)),
            pl.BlockSpec((_EDIM, _WFEAT), lambda i: (0, 0)),
        ],
        out_specs=pl.BlockSpec((_BW, 2 * _WFEAT), lambda i: (i, 0)),
        out_shape=jax.ShapeDtypeStruct((_HV, 2 * _WFEAT), jnp.float32),
        compiler_params=pltpu.CompilerParams(
            dimension_semantics=("arbitrary",)),
    )(emb, W)


def _sc_gather_body(table_hbm, idx_hbm, out_hbm, idx_v, rows_v, sem):
    wid = lax.axis_index("s") * _NC + lax.axis_index("c")
    base = wid * _BPW
    pltpu.sync_copy(idx_hbm.at[pl.ds(base, _BPW)], idx_v)
    pltpu.async_copy(table_hbm.at[idx_v], rows_v, sem).wait()
    pltpu.sync_copy(rows_v, out_hbm.at[pl.ds(base, _BPW)])


def _sc_gather(table2, idx_mod):
    mesh = plsc.VectorSubcoreMesh(core_axis_name="c", subcore_axis_name="s")
    fn = functools.partial(
        pl.kernel,
        mesh=mesh,
        out_type=jax.ShapeDtypeStruct((_N, 2 * _WFEAT), jnp.float32),
        scratch_types=[
            pltpu.VMEM((_BPW,), jnp.int32),
            pltpu.VMEM((_BPW, 2 * _WFEAT), jnp.float32),
            pltpu.SemaphoreType.DMA,
        ],
    )(_sc_gather_body)
    return fn(table2, idx_mod)


def _tc_body(adj_ref, pairs_ref, half_ref, a1_ref, a2r_ref, wc_ref,
             bc_ref, att_ref, sent_ref, pool_ref, label_ref, wh_s, f2r_s,
             pool_s):
    i = pl.program_id(0)

    @pl.when(i == 0)
    def _init():
        pairs = pairs_ref[...]
        wh = jnp.where(half_ref[...] > 0,
                       lax.slice(pairs, (0, _WFEAT), (_N, 2 * _WFEAT)),
                       lax.slice(pairs, (0, 0), (_N, _WFEAT)))
        wh_s[...] = wh
        f2r_s[...] = lax.dot_general(
            a2r_ref[...], wh, (((1,), (1,)), ((), ())),
            preferred_element_type=jnp.float32)
        pool_s[...] = jnp.zeros_like(pool_s)

    row0 = pl.multiple_of(i * _BR, _BR)
    wh_strip = wh_s[pl.ds(row0, _BR), :]
    f1 = jnp.dot(wh_strip, a1_ref[...],
                 preferred_element_type=jnp.float32)
    e = f1 + f2r_s[...]
    e = jnp.maximum(e, _SLOPE * e)
    e = jnp.where(adj_ref[...] > 0, e, jnp.float32(-9e15))
    m = jnp.max(e, axis=1, keepdims=True)
    p = jnp.exp(e - m)
    s = jnp.sum(p, axis=1, keepdims=True)
    att = p / s
    att_ref[...] = att
    h = jnp.dot(att, wh_s[...], preferred_element_type=jnp.float32)
    sent = jnp.where(h > 0, h, jnp.exp(h) - 1.0)
    sent_ref[...] = sent
    pool_s[...] += jnp.sum(sent, axis=0, keepdims=True)

    @pl.when(i == _NSTRIPS - 1)
    def _fin():
        pool = pool_s[...] / jnp.float32(_N)
        pool_ref[...] = pool
        logits = jnp.dot(pool, wc_ref[...],
                         preferred_element_type=jnp.float32) + bc_ref[...]
        lm = jnp.max(logits, axis=1, keepdims=True)
        ex = jnp.exp(logits - lm)
        label_ref[...] = ex / jnp.sum(ex, axis=1, keepdims=True)


def _tc_main(adj, pairs, half, a1, a2r, Wc, bcr):
    return pl.pallas_call(
        _tc_body,
        grid=(_NSTRIPS,),
        in_specs=[
            pl.BlockSpec((_BR, _N), lambda i: (i, 0)),
            pl.BlockSpec((_N, 2 * _WFEAT), lambda i: (0, 0)),
            pl.BlockSpec((_N, 1), lambda i: (0, 0)),
            pl.BlockSpec((_WFEAT, 1), lambda i: (0, 0)),
            pl.BlockSpec((1, _WFEAT), lambda i: (0, 0)),
            pl.BlockSpec((_WFEAT, _LABELS), lambda i: (0, 0)),
            pl.BlockSpec((1, _LABELS), lambda i: (0, 0)),
        ],
        out_specs=[
            pl.BlockSpec((_BR, _N), lambda i: (i, 0)),
            pl.BlockSpec((_BR, _WFEAT), lambda i: (i, 0)),
            pl.BlockSpec((1, _WFEAT), lambda i: (0, 0)),
            pl.BlockSpec((1, _LABELS), lambda i: (0, 0)),
        ],
        out_shape=[
            jax.ShapeDtypeStruct((_N, _N), jnp.float32),
            jax.ShapeDtypeStruct((_N, _WFEAT), jnp.float32),
            jax.ShapeDtypeStruct((1, _WFEAT), jnp.float32),
            jax.ShapeDtypeStruct((1, _LABELS), jnp.float32),
        ],
        scratch_shapes=[
            pltpu.VMEM((_N, _WFEAT), jnp.float32),
            pltpu.VMEM((1, _N), jnp.float32),
            pltpu.VMEM((1, _WFEAT), jnp.float32),
        ],
        compiler_params=pltpu.CompilerParams(
            dimension_semantics=("arbitrary",)),
    )(adj, pairs, half, a1, a2r, Wc, bcr)


def kernel(inSen, adj, emb, W, a, Wc, bc):
    idx = inSen.astype(jnp.int32)
    table2 = _wh_all(emb, W)
    blk = idx // (2 * _BW)
    loc = idx % (2 * _BW)
    pairs = _sc_gather(table2, blk * _BW + loc % _BW)
    half = (loc // _BW).astype(jnp.float32).reshape(_N, 1)
    a1 = a[:_WFEAT]
    a2r = a[_WFEAT:].reshape(1, _WFEAT)
    bcr = bc.reshape(1, _LABELS)
    att, sent, pool, label = _tc_main(adj, pairs, half, a1, a2r, Wc, bcr)
    return (pool.reshape(_WFEAT), att, sent, label.reshape(_LABELS))


# speedup vs baseline: 1.5496x; 1.3320x over previous
import functools

import jax
import jax.numpy as jnp
from jax import lax
from jax.experimental import pallas as pl
from jax.experimental.pallas import tpu as pltpu
from jax.experimental.pallas import tpu_sc as plsc

_N = 4096
_VOCAB = 100000
_EDIM = 64
_WFEAT = 64
_LABELS = 2
_SLOPE = 0.01
_BR = 256
_NSTRIPS = _N // _BR

_NC = 2
_NS = 16
_NW = _NC * _NS
_BPW = _N // _NW
_L = 16


def _sc_gather_body(table_hbm, idx_hbm, out_hbm, idx_v, rows_v, sem):
    wid = lax.axis_index("s") * _NC + lax.axis_index("c")
    base = wid * _BPW
    pltpu.sync_copy(idx_hbm.at[pl.ds(base, _BPW)], idx_v)
    lanes = lax.iota(jnp.int32, _L)

    def _fire(g, carry):
        v = idx_v[pl.ds(g * _L, _L)]
        for l in range(_L):
            r = jnp.sum(jnp.where(lanes == l, v, 0))
            pltpu.make_async_copy(
                table_hbm.at[pl.ds(r, 1), :],
                rows_v.at[pl.ds(g * _L + l, 1), :], sem).start()
        return carry

    lax.fori_loop(0, _BPW // _L, _fire, 0)

    def _drain(g, carry):
        pltpu.make_async_copy(
            table_hbm.at[pl.ds(0, 1), :],
            rows_v.at[pl.ds(0, 1), :], sem).wait()
        return carry

    lax.fori_loop(0, _BPW, _drain, 0)
    pltpu.sync_copy(rows_v, out_hbm.at[pl.ds(base, _BPW)])


def _sc_gather(table, idx):
    mesh = plsc.VectorSubcoreMesh(core_axis_name="c", subcore_axis_name="s")
    fn = functools.partial(
        pl.kernel,
        mesh=mesh,
        out_type=jax.ShapeDtypeStruct((_N, _EDIM), jnp.float32),
        scratch_types=[
            pltpu.VMEM((_BPW,), jnp.int32),
            pltpu.VMEM((_BPW, _EDIM), jnp.float32),
            pltpu.SemaphoreType.DMA,
        ],
        compiler_params=pltpu.CompilerParams(needs_layout_passes=False),
    )(_sc_gather_body)
    return fn(table, idx)


def _tc_body(adj_ref, words_ref, w_ref, a1_ref, a2r_ref, wc_ref, bc_ref,
             att_ref, sent_ref, pool_ref, label_ref, wh_s, f2r_s, pool_s):
    i = pl.program_id(0)

    @pl.when(i == 0)
    def _init():
        wh = jnp.dot(words_ref[...], w_ref[...],
                     preferred_element_type=jnp.float32)
        wh_s[...] = wh
        f2r_s[...] = lax.dot_general(
            a2r_ref[...], wh, (((1,), (1,)), ((), ())),
            preferred_element_type=jnp.float32)
        pool_s[...] = jnp.zeros_like(pool_s)

    row0 = pl.multiple_of(i * _BR, _BR)
    wh_strip = wh_s[pl.ds(row0, _BR), :]
    f1 = jnp.dot(wh_strip, a1_ref[...],
                 preferred_element_type=jnp.float32)
    e = f1 + f2r_s[...]
    e = jnp.maximum(e, _SLOPE * e)
    e = jnp.where(adj_ref[...] > 0, e, jnp.float32(-9e15))
    m = jnp.max(e, axis=1, keepdims=True)
    p = jnp.exp(e - m)
    s = jnp.sum(p, axis=1, keepdims=True)
    att = p / s
    att_ref[...] = att
    h = jnp.dot(att, wh_s[...], preferred_element_type=jnp.float32)
    sent = jnp.where(h > 0, h, jnp.exp(h) - 1.0)
    sent_ref[...] = sent
    pool_s[...] += jnp.sum(sent, axis=0, keepdims=True)

    @pl.when(i == _NSTRIPS - 1)
    def _fin():
        pool = pool_s[...] / jnp.float32(_N)
        pool_ref[...] = pool
        logits = jnp.dot(pool, wc_ref[...],
                         preferred_element_type=jnp.float32) + bc_ref[...]
        lm = jnp.max(logits, axis=1, keepdims=True)
        ex = jnp.exp(logits - lm)
        label_ref[...] = ex / jnp.sum(ex, axis=1, keepdims=True)


def _tc_main(adj, words, W, a1, a2r, Wc, bcr):
    return pl.pallas_call(
        _tc_body,
        grid=(_NSTRIPS,),
        in_specs=[
            pl.BlockSpec((_BR, _N), lambda i: (i, 0)),
            pl.BlockSpec((_N, _EDIM), lambda i: (0, 0)),
            pl.BlockSpec((_EDIM, _WFEAT), lambda i: (0, 0)),
            pl.BlockSpec((_WFEAT, 1), lambda i: (0, 0)),
            pl.BlockSpec((1, _WFEAT), lambda i: (0, 0)),
            pl.BlockSpec((_WFEAT, _LABELS), lambda i: (0, 0)),
            pl.BlockSpec((1, _LABELS), lambda i: (0, 0)),
        ],
        out_specs=[
            pl.BlockSpec((_BR, _N), lambda i: (i, 0)),
            pl.BlockSpec((_BR, _WFEAT), lambda i: (i, 0)),
            pl.BlockSpec((1, _WFEAT), lambda i: (0, 0)),
            pl.BlockSpec((1, _LABELS), lambda i: (0, 0)),
        ],
        out_shape=[
            jax.ShapeDtypeStruct((_N, _N), jnp.float32),
            jax.ShapeDtypeStruct((_N, _WFEAT), jnp.float32),
            jax.ShapeDtypeStruct((1, _WFEAT), jnp.float32),
            jax.ShapeDtypeStruct((1, _LABELS), jnp.float32),
        ],
        scratch_shapes=[
            pltpu.VMEM((_N, _WFEAT), jnp.float32),
            pltpu.VMEM((1, _N), jnp.float32),
            pltpu.VMEM((1, _WFEAT), jnp.float32),
        ],
        compiler_params=pltpu.CompilerParams(
            dimension_semantics=("arbitrary",)),
    )(adj, words, W, a1, a2r, Wc, bcr)


def kernel(inSen, adj, emb, W, a, Wc, bc):
    idx = inSen.astype(jnp.int32)
    words = _sc_gather(emb, idx)
    a1 = a[:_WFEAT]
    a2r = a[_WFEAT:].reshape(1, _WFEAT)
    bcr = bc.reshape(1, _LABELS)
    att, sent, pool, label = _tc_main(adj, words, W, a1, a2r, Wc, bcr)
    return (pool.reshape(_WFEAT), att, sent, label.reshape(_LABELS))


# speedup vs baseline: 1.6622x; 1.0726x over previous
import functools

import jax
import jax.numpy as jnp
from jax import lax
from jax.experimental import pallas as pl
from jax.experimental.pallas import tpu as pltpu
from jax.experimental.pallas import tpu_sc as plsc

_N = 4096
_VOCAB = 100000
_EDIM = 64
_WFEAT = 64
_LABELS = 2
_SLOPE = 0.01
_HV = _VOCAB // 2
_BR = 256
_NSTRIPS = _N // _BR

_NC = 2
_NS = 16
_NW = _NC * _NS
_BPW = _N // _NW
_L = 16


def _sc_gather_body(table_hbm, idx_hbm, out_hbm, idx_v, rows_v, sem):
    wid = lax.axis_index("s") * _NC + lax.axis_index("c")
    base = wid * _BPW
    pltpu.sync_copy(idx_hbm.at[pl.ds(base, _BPW)], idx_v)
    lanes = lax.iota(jnp.int32, _L)

    def _fire(g, carry):
        v = idx_v[pl.ds(g * _L, _L)]
        vd = v // _HV
        vq = v % _HV
        for l in range(_L):
            d = jnp.sum(jnp.where(lanes == l, vd, 0))
            q = jnp.sum(jnp.where(lanes == l, vq, 0))
            pltpu.make_async_copy(
                table_hbm.at[d, pl.ds(q, 1), :],
                rows_v.at[pl.ds(g * _L + l, 1), :], sem).start()
        return carry

    lax.fori_loop(0, _BPW // _L, _fire, 0)

    def _drain(g, carry):
        pltpu.make_async_copy(
            table_hbm.at[0, pl.ds(0, 1), :],
            rows_v.at[pl.ds(0, 1), :], sem).wait()
        return carry

    lax.fori_loop(0, _BPW, _drain, 0)
    pltpu.sync_copy(rows_v, out_hbm.at[pl.ds(base, _BPW)])


def _sc_gather(table3, idx):
    mesh = plsc.VectorSubcoreMesh(core_axis_name="c", subcore_axis_name="s")
    fn = functools.partial(
        pl.kernel,
        mesh=mesh,
        out_type=jax.ShapeDtypeStruct((_N, _EDIM), jnp.float32),
        scratch_types=[
            pltpu.VMEM((_BPW,), jnp.int32),
            pltpu.VMEM((_BPW, _EDIM), jnp.float32),
            pltpu.SemaphoreType.DMA,
        ],
        compiler_params=pltpu.CompilerParams(needs_layout_passes=False),
    )(_sc_gather_body)
    return fn(table3, idx)


def _tc_body(adj_ref, words_ref, w_ref, a1_ref, a2r_ref, wc_ref, bc_ref,
             att_ref, sent_ref, pool_ref, label_ref, wh_s, f2r_s, pool_s):
    i = pl.program_id(0)

    @pl.when(i == 0)
    def _init():
        wh = jnp.dot(words_ref[...], w_ref[...],
                     preferred_element_type=jnp.float32)
        wh_s[...] = wh
        f2r_s[...] = lax.dot_general(
            a2r_ref[...], wh, (((1,), (1,)), ((), ())),
            preferred_element_type=jnp.float32)
        pool_s[...] = jnp.zeros_like(pool_s)

    row0 = pl.multiple_of(i * _BR, _BR)
    wh_strip = wh_s[pl.ds(row0, _BR), :]
    f1 = jnp.dot(wh_strip, a1_ref[...],
                 preferred_element_type=jnp.float32)
    e = f1 + f2r_s[...]
    e = jnp.maximum(e, _SLOPE * e)
    e = jnp.where(adj_ref[...] > 0, e, jnp.float32(-9e15))
    m = jnp.max(e, axis=1, keepdims=True)
    p = jnp.exp(e - m)
    s = jnp.sum(p, axis=1, keepdims=True)
    att = p / s
    att_ref[...] = att
    h = jnp.dot(att, wh_s[...], preferred_element_type=jnp.float32)
    sent = jnp.where(h > 0, h, jnp.exp(h) - 1.0)
    sent_ref[...] = sent
    pool_s[...] += jnp.sum(sent, axis=0, keepdims=True)

    @pl.when(i == _NSTRIPS - 1)
    def _fin():
        pool = pool_s[...] / jnp.float32(_N)
        pool_ref[...] = pool
        logits = jnp.dot(pool, wc_ref[...],
                         preferred_element_type=jnp.float32) + bc_ref[...]
        lm = jnp.max(logits, axis=1, keepdims=True)
        ex = jnp.exp(logits - lm)
        label_ref[...] = ex / jnp.sum(ex, axis=1, keepdims=True)


def _tc_main(adj, words, W, a1, a2r, Wc, bcr):
    return pl.pallas_call(
        _tc_body,
        grid=(_NSTRIPS,),
        in_specs=[
            pl.BlockSpec((_BR, _N), lambda i: (i, 0)),
            pl.BlockSpec((_N, _EDIM), lambda i: (0, 0)),
            pl.BlockSpec((_EDIM, _WFEAT), lambda i: (0, 0)),
            pl.BlockSpec((_WFEAT, 1), lambda i: (0, 0)),
            pl.BlockSpec((1, _WFEAT), lambda i: (0, 0)),
            pl.BlockSpec((_WFEAT, _LABELS), lambda i: (0, 0)),
            pl.BlockSpec((1, _LABELS), lambda i: (0, 0)),
        ],
        out_specs=[
            pl.BlockSpec((_BR, _N), lambda i: (i, 0)),
            pl.BlockSpec((_BR, _WFEAT), lambda i: (i, 0)),
            pl.BlockSpec((1, _WFEAT), lambda i: (0, 0)),
            pl.BlockSpec((1, _LABELS), lambda i: (0, 0)),
        ],
        out_shape=[
            jax.ShapeDtypeStruct((_N, _N), jnp.float32),
            jax.ShapeDtypeStruct((_N, _WFEAT), jnp.float32),
            jax.ShapeDtypeStruct((1, _WFEAT), jnp.float32),
            jax.ShapeDtypeStruct((1, _LABELS), jnp.float32),
        ],
        scratch_shapes=[
            pltpu.VMEM((_N, _WFEAT), jnp.float32),
            pltpu.VMEM((1, _N), jnp.float32),
            pltpu.VMEM((1, _WFEAT), jnp.float32),
        ],
        compiler_params=pltpu.CompilerParams(
            dimension_semantics=("arbitrary",)),
    )(adj, words, W, a1, a2r, Wc, bcr)


def kernel(inSen, adj, emb, W, a, Wc, bc):
    idx = inSen.astype(jnp.int32)
    words = _sc_gather(emb.reshape(2, _HV, _EDIM), idx)
    a1 = a[:_WFEAT]
    a2r = a[_WFEAT:].reshape(1, _WFEAT)
    bcr = bc.reshape(1, _LABELS)
    att, sent, pool, label = _tc_main(adj, words, W, a1, a2r, Wc, bcr)
    return (pool.reshape(_WFEAT), att, sent, label.reshape(_LABELS))


# speedup vs baseline: 1.6691x; 1.0042x over previous
import functools

import jax
import jax.numpy as jnp
from jax import lax
from jax.experimental import pallas as pl
from jax.experimental.pallas import tpu as pltpu
from jax.experimental.pallas import tpu_sc as plsc

_N = 4096
_VOCAB = 100000
_EDIM = 64
_WFEAT = 64
_LABELS = 2
_SLOPE = 0.01
_HV = _VOCAB // 2
_BR = 512
_NSTRIPS = _N // _BR

_NC = 2
_NS = 16
_NW = _NC * _NS
_BPW = _N // _NW
_L = 16


def _sc_gather_body(table_hbm, idx_hbm, out_hbm, idx_v, rows_v, sem):
    wid = lax.axis_index("s") * _NC + lax.axis_index("c")
    base = wid * _BPW
    pltpu.sync_copy(idx_hbm.at[pl.ds(base, _BPW)], idx_v)
    lanes = lax.iota(jnp.int32, _L)

    def _fire(g, carry):
        v = idx_v[pl.ds(g * _L, _L)]
        vd = v // _HV
        vq = v % _HV
        for l in range(_L):
            d = jnp.sum(jnp.where(lanes == l, vd, 0))
            q = jnp.sum(jnp.where(lanes == l, vq, 0))
            pltpu.make_async_copy(
                table_hbm.at[d, pl.ds(q, 1), :],
                rows_v.at[pl.ds(g * _L + l, 1), :], sem).start()
        return carry

    lax.fori_loop(0, _BPW // _L, _fire, 0)

    def _drain(g, carry):
        pltpu.make_async_copy(
            table_hbm.at[0, pl.ds(0, 1), :],
            rows_v.at[pl.ds(0, 1), :], sem).wait()
        return carry

    lax.fori_loop(0, _BPW, _drain, 0)
    pltpu.sync_copy(rows_v, out_hbm.at[pl.ds(base, _BPW)])


def _sc_gather(table3, idx):
    mesh = plsc.VectorSubcoreMesh(core_axis_name="c", subcore_axis_name="s")
    fn = functools.partial(
        pl.kernel,
        mesh=mesh,
        out_type=jax.ShapeDtypeStruct((_N, _EDIM), jnp.float32),
        scratch_types=[
            pltpu.VMEM((_BPW,), jnp.int32),
            pltpu.VMEM((_BPW, _EDIM), jnp.float32),
            pltpu.SemaphoreType.DMA,
        ],
        compiler_params=pltpu.CompilerParams(needs_layout_passes=False),
    )(_sc_gather_body)
    return fn(table3, idx)


def _tc_body(adj_ref, words_ref, w_ref, a1_ref, a2r_ref, wc_ref, bc_ref,
             att_ref, sent_ref, pool_ref, label_ref, wh_s, f2r_s, pool_s):
    i = pl.program_id(0)

    @pl.when(i == 0)
    def _init():
        wh = jnp.dot(words_ref[...], w_ref[...],
                     preferred_element_type=jnp.float32)
        wh_s[...] = wh
        f2r_s[...] = lax.dot_general(
            a2r_ref[...], wh, (((1,), (1,)), ((), ())),
            preferred_element_type=jnp.float32)
        pool_s[...] = jnp.zeros_like(pool_s)

    row0 = pl.multiple_of(i * _BR, _BR)
    wh_strip = wh_s[pl.ds(row0, _BR), :]
    f1 = jnp.dot(wh_strip, a1_ref[...],
                 preferred_element_type=jnp.float32)
    e = f1 + f2r_s[...]
    e = jnp.maximum(e, _SLOPE * e)
    e = jnp.where(adj_ref[...] > 0, e, jnp.float32(-9e15))
    m = jnp.max(e, axis=1, keepdims=True)
    p = jnp.exp(e - m)
    s = jnp.sum(p, axis=1, keepdims=True)
    att = p / s
    att_ref[...] = att
    h = jnp.dot(att, wh_s[...], preferred_element_type=jnp.float32)
    sent = jnp.where(h > 0, h, jnp.exp(h) - 1.0)
    sent_ref[...] = sent
    pool_s[...] += jnp.sum(sent, axis=0, keepdims=True)

    @pl.when(i == _NSTRIPS - 1)
    def _fin():
        pool = pool_s[...] / jnp.float32(_N)
        pool_ref[...] = pool
        logits = jnp.dot(pool, wc_ref[...],
                         preferred_element_type=jnp.float32) + bc_ref[...]
        lm = jnp.max(logits, axis=1, keepdims=True)
        ex = jnp.exp(logits - lm)
        label_ref[...] = ex / jnp.sum(ex, axis=1, keepdims=True)


def _tc_main(adj, words, W, a1, a2r, Wc, bcr):
    return pl.pallas_call(
        _tc_body,
        grid=(_NSTRIPS,),
        in_specs=[
            pl.BlockSpec((_BR, _N), lambda i: (i, 0)),
            pl.BlockSpec((_N, _EDIM), lambda i: (0, 0)),
            pl.BlockSpec((_EDIM, _WFEAT), lambda i: (0, 0)),
            pl.BlockSpec((_WFEAT, 1), lambda i: (0, 0)),
            pl.BlockSpec((1, _WFEAT), lambda i: (0, 0)),
            pl.BlockSpec((_WFEAT, _LABELS), lambda i: (0, 0)),
            pl.BlockSpec((1, _LABELS), lambda i: (0, 0)),
        ],
        out_specs=[
            pl.BlockSpec((_BR, _N), lambda i: (i, 0)),
            pl.BlockSpec((_BR, _WFEAT), lambda i: (i, 0)),
            pl.BlockSpec((1, _WFEAT), lambda i: (0, 0)),
            pl.BlockSpec((1, _LABELS), lambda i: (0, 0)),
        ],
        out_shape=[
            jax.ShapeDtypeStruct((_N, _N), jnp.float32),
            jax.ShapeDtypeStruct((_N, _WFEAT), jnp.float32),
            jax.ShapeDtypeStruct((1, _WFEAT), jnp.float32),
            jax.ShapeDtypeStruct((1, _LABELS), jnp.float32),
        ],
        scratch_shapes=[
            pltpu.VMEM((_N, _WFEAT), jnp.float32),
            pltpu.VMEM((1, _N), jnp.float32),
            pltpu.VMEM((1, _WFEAT), jnp.float32),
        ],
        compiler_params=pltpu.CompilerParams(
            dimension_semantics=("arbitrary",),
            vmem_limit_bytes=100 * 1024 * 1024),
    )(adj, words, W, a1, a2r, Wc, bcr)


def kernel(inSen, adj, emb, W, a, Wc, bc):
    idx = inSen.astype(jnp.int32)
    words = _sc_gather(emb.reshape(2, _HV, _EDIM), idx)
    a1 = a[:_WFEAT]
    a2r = a[_WFEAT:].reshape(1, _WFEAT)
    bcr = bc.reshape(1, _LABELS)
    att, sent, pool, label = _tc_main(adj, words, W, a1, a2r, Wc, bcr)
    return (pool.reshape(_WFEAT), att, sent, label.reshape(_LABELS))
